# baseline (speedup 1.0000x reference)
import functools

import jax
import jax.numpy as jnp
from jax import lax
from jax.experimental import pallas as pl
from jax.experimental.pallas import tpu as pltpu
from jax.experimental.pallas import tpu_sc as plsc

_D = 64


@functools.lru_cache(maxsize=None)
def _build_gather(B0, H, R):
    info = plsc.get_sparse_core_info()
    NC, NS = info.num_cores, info.num_subcores
    NW = NC * NS
    assert B0 % (NW * R) == 0
    rows_per_w = B0 // NW
    nt = rows_per_w // R
    assert nt % 2 == 0
    mesh = plsc.VectorSubcoreMesh(core_axis_name="c", subcore_axis_name="s")

    @functools.partial(
        pl.kernel,
        mesh=mesh,
        out_type=jax.ShapeDtypeStruct((B0, H, _D), jnp.float32),
        scratch_types=[
            pltpu.VMEM((2, R, H), jnp.int32),
            pltpu.VMEM((2, R, H, _D), jnp.float32),
            pltpu.SemaphoreType.DMA,
            pltpu.SemaphoreType.DMA,
            pltpu.SemaphoreType.DMA,
            pltpu.SemaphoreType.DMA,
        ],
        compiler_params=pltpu.CompilerParams(use_tc_tiling_on_sc=False),
    )
    def gather_kernel(idx_hbm, table_hbm, out_hbm, idx_v, rows_v, g0, g1, o0, o1):
        gsem = (g0, g1)
        osem = (o0, o1)
        wid = lax.axis_index("s") * NC + lax.axis_index("c")
        base = wid * rows_per_w

        def idx_load(b, r0):
            pltpu.sync_copy(idx_hbm.at[pl.ds(r0, R)], idx_v.at[b])

        def gather_descs(b):
            return [
                pltpu.make_async_copy(
                    table_hbm.at[idx_v.at[b, r]], rows_v.at[b, r], gsem[b])
                for r in range(R)
            ]

        def out_desc(b, r0):
            return pltpu.make_async_copy(
                rows_v.at[b], out_hbm.at[pl.ds(r0, R)], osem[b])

        for b in range(2):
            idx_load(b, base + b * R)
            for desc in gather_descs(b):
                desc.start()

        npairs = nt // 2

        def body(tt, carry):
            for b in range(2):
                r0 = base + (tt * 2 + b) * R
                for desc in gather_descs(b):
                    desc.wait()
                out_desc(b, r0).start()
                idx_load(b, r0 + 2 * R)
                out_desc(b, r0).wait()
                for desc in gather_descs(b):
                    desc.start()
            return carry

        lax.fori_loop(0, npairs - 1, body, 0)

        for b in range(2):
            r0 = base + ((npairs - 1) * 2 + b) * R
            for desc in gather_descs(b):
                desc.wait()
            out_desc(b, r0).start()
        for b in range(2):
            r0 = base + ((npairs - 1) * 2 + b) * R
            out_desc(b, r0).wait()

    return gather_kernel


def kernel(indices, table):
    B0, H = indices.shape
    return _build_gather(B0, H, 16)(indices.astype(jnp.int32), table)


# speedup vs baseline: 1.7898x; 1.7898x over previous
import functools

import jax
import jax.numpy as jnp
from jax import lax
from jax.experimental import pallas as pl
from jax.experimental.pallas import tpu as pltpu
from jax.experimental.pallas import tpu_sc as plsc

_D = 64
_HP = 56
_DP = 128


@functools.lru_cache(maxsize=None)
def _build_gather(B0, H, R):
    info = plsc.get_sparse_core_info()
    NC, NS = info.num_cores, info.num_subcores
    NW = NC * NS
    assert B0 % (NW * R) == 0
    rows_per_w = B0 // NW
    nt = rows_per_w // R
    assert nt % 2 == 0
    mesh = plsc.VectorSubcoreMesh(core_axis_name="c", subcore_axis_name="s")

    @functools.partial(
        pl.kernel,
        mesh=mesh,
        out_type=jax.ShapeDtypeStruct((B0, _HP, _DP), jnp.float32),
        scratch_types=[
            pltpu.VMEM((2, R, H), jnp.int32),
            pltpu.VMEM((2, R, H, _D), jnp.float32),
            pltpu.SemaphoreType.DMA,
            pltpu.SemaphoreType.DMA,
            pltpu.SemaphoreType.DMA,
            pltpu.SemaphoreType.DMA,
        ],
        compiler_params=pltpu.CompilerParams(use_tc_tiling_on_sc=False),
    )
    def gather_kernel(idx_hbm, table_hbm, out_hbm, idx_v, rows_v, g0, g1, o0, o1):
        gsem = (g0, g1)
        osem = (o0, o1)
        wid = lax.axis_index("s") * NC + lax.axis_index("c")
        base = wid * rows_per_w

        def idx_load(b, r0):
            pltpu.sync_copy(idx_hbm.at[pl.ds(r0, R)], idx_v.at[b])

        def gather_descs(b):
            return [
                pltpu.make_async_copy(
                    table_hbm.at[idx_v.at[b, r]], rows_v.at[b, r], gsem[b])
                for r in range(R)
            ]

        def out_desc(b, r0):
            return pltpu.make_async_copy(
                rows_v.at[b],
                out_hbm.at[pl.ds(r0, R), pl.ds(0, H), pl.ds(0, _D)],
                osem[b])

        for b in range(2):
            idx_load(b, base + b * R)
            for desc in gather_descs(b):
                desc.start()

        npairs = nt // 2

        def body(tt, carry):
            for b in range(2):
                r0 = base + (tt * 2 + b) * R
                for desc in gather_descs(b):
                    desc.wait()
                out_desc(b, r0).start()
                idx_load(b, r0 + 2 * R)
                out_desc(b, r0).wait()
                for desc in gather_descs(b):
                    desc.start()
            return carry

        lax.fori_loop(0, npairs - 1, body, 0)

        for b in range(2):
            r0 = base + ((npairs - 1) * 2 + b) * R
            for desc in gather_descs(b):
                desc.wait()
            out_desc(b, r0).start()
        for b in range(2):
            r0 = base + ((npairs - 1) * 2 + b) * R
            out_desc(b, r0).wait()

    return gather_kernel


def kernel(indices, table):
    B0, H = indices.shape
    padded = _build_gather(B0, H, 16)(indices.astype(jnp.int32), table)
    return padded[:, :H, :_D]


# speedup vs baseline: 1.7901x; 1.0002x over previous
import functools

import jax
import jax.numpy as jnp
from jax import lax
from jax.experimental import pallas as pl
from jax.experimental.pallas import tpu as pltpu
from jax.experimental.pallas import tpu_sc as plsc

_D = 64
_HP = 56
_DP = 128


@functools.lru_cache(maxsize=None)
def _build_gather(B0, H, R):
    info = plsc.get_sparse_core_info()
    NC, NS = info.num_cores, info.num_subcores
    NW = NC * NS
    assert B0 % (NW * R) == 0
    rows_per_w = B0 // NW
    nt = rows_per_w // R
    assert nt % 2 == 0
    mesh = plsc.VectorSubcoreMesh(core_axis_name="c", subcore_axis_name="s")

    @functools.partial(
        pl.kernel,
        mesh=mesh,
        out_type=jax.ShapeDtypeStruct((B0, _HP, _DP), jnp.float32),
        scratch_types=[
            pltpu.VMEM((2, R, H), jnp.int32),
            pltpu.VMEM((2, R, H, _D), jnp.float32),
            pltpu.SemaphoreType.DMA,
            pltpu.SemaphoreType.DMA,
            pltpu.SemaphoreType.DMA,
            pltpu.SemaphoreType.DMA,
            pltpu.SemaphoreType.DMA,
            pltpu.SemaphoreType.DMA,
        ],
        compiler_params=pltpu.CompilerParams(use_tc_tiling_on_sc=False),
    )
    def gather_kernel(idx_hbm, table_hbm, out_hbm, idx_v, rows_v,
                      g0, g1, o0, o1, i0, i1):
        gsem = (g0, g1)
        osem = (o0, o1)
        isem = (i0, i1)
        wid = lax.axis_index("s") * NC + lax.axis_index("c")
        base = wid * rows_per_w

        def idx_desc(b, r0):
            return pltpu.make_async_copy(
                idx_hbm.at[pl.ds(r0, R)], idx_v.at[b], isem[b])

        def gather_descs(b):
            return [
                pltpu.make_async_copy(
                    table_hbm.at[idx_v.at[b, r]], rows_v.at[b, r], gsem[b])
                for r in range(R)
            ]

        def out_desc(b, r0):
            return pltpu.make_async_copy(
                rows_v.at[b],
                out_hbm.at[pl.ds(r0, R), pl.ds(0, H), pl.ds(0, _D)],
                osem[b])

        for b in range(2):
            idx_desc(b, base + b * R).start()
        for b in range(2):
            idx_desc(b, base + b * R).wait()
            for desc in gather_descs(b):
                desc.start()

        npairs = nt // 2

        def body(tt, carry):
            for b in range(2):
                r0 = base + (tt * 2 + b) * R
                for desc in gather_descs(b):
                    desc.wait()
                out_desc(b, r0).start()
                idx_desc(b, r0 + 2 * R).start()
                out_desc(b, r0).wait()
                idx_desc(b, r0 + 2 * R).wait()
                for desc in gather_descs(b):
                    desc.start()
            return carry

        lax.fori_loop(0, npairs - 1, body, 0)

        for b in range(2):
            r0 = base + ((npairs - 1) * 2 + b) * R
            for desc in gather_descs(b):
                desc.wait()
            out_desc(b, r0).start()
        for b in range(2):
            r0 = base + ((npairs - 1) * 2 + b) * R
            out_desc(b, r0).wait()

    return gather_kernel


def kernel(indices, table):
    B0, H = indices.shape
    padded = _build_gather(B0, H, 16)(indices.astype(jnp.int32), table)
    return padded[:, :H, :_D]


# speedup vs baseline: 1.8455x; 1.0309x over previous
import functools

import jax
import jax.numpy as jnp
from jax import lax
from jax.experimental import pallas as pl
from jax.experimental.pallas import tpu as pltpu
from jax.experimental.pallas import tpu_sc as plsc

_D = 64
_HP = 56
_DP = 128


@functools.lru_cache(maxsize=None)
def _build_gather(B0, H):
    info = plsc.get_sparse_core_info()
    NC, NS = info.num_cores, info.num_subcores
    NW = NC * NS
    assert B0 % NW == 0 and H % 2 == 0
    bw = B0 // NW
    mesh = plsc.VectorSubcoreMesh(core_axis_name="c", subcore_axis_name="s")

    @functools.partial(
        pl.kernel,
        mesh=mesh,
        out_type=jax.ShapeDtypeStruct((B0, _HP, _DP), jnp.float32),
        scratch_types=[
            pltpu.VMEM((H, bw), jnp.int32),
            pltpu.VMEM((2, bw, _D), jnp.float32),
            pltpu.SemaphoreType.DMA,
            pltpu.SemaphoreType.DMA,
            pltpu.SemaphoreType.DMA,
            pltpu.SemaphoreType.DMA,
        ],
        compiler_params=pltpu.CompilerParams(use_tc_tiling_on_sc=False),
    )
    def gather_kernel(idxt_hbm, table_hbm, out_hbm, idx_v, rows_v,
                      g0, g1, o0, o1):
        gsem = (g0, g1)
        osem = (o0, o1)
        wid = lax.axis_index("s") * NC + lax.axis_index("c")
        b0 = wid * bw

        def gather_desc(b, h):
            return pltpu.make_async_copy(
                table_hbm.at[idx_v.at[h]], rows_v.at[b], gsem[b])

        def out_desc(b, h):
            return pltpu.make_async_copy(
                rows_v.at[b],
                out_hbm.at[pl.ds(b0, bw), h, pl.ds(0, _D)],
                osem[b])

        pltpu.sync_copy(idxt_hbm.at[pl.ds(0, H), pl.ds(b0, bw)], idx_v)
        for b in range(2):
            gather_desc(b, b).start()

        npairs = H // 2

        def body(tt, carry):
            for b in range(2):
                h = tt * 2 + b
                gather_desc(b, h).wait()
                out_desc(b, h).start()
                out_desc(b, h).wait()
                gather_desc(b, h + 2).start()
            return carry

        lax.fori_loop(0, npairs - 1, body, 0)

        for b in range(2):
            h = (npairs - 1) * 2 + b
            gather_desc(b, h).wait()
            out_desc(b, h).start()
        for b in range(2):
            h = (npairs - 1) * 2 + b
            out_desc(b, h).wait()

    return gather_kernel


def kernel(indices, table):
    B0, H = indices.shape
    idxt = indices.astype(jnp.int32).T
    padded = _build_gather(B0, H)(idxt, table)
    return padded[:, :H, :_D]
